# Initial kernel scaffold; baseline (speedup 1.0000x reference)
import jax
import jax.numpy as jnp
from jax.experimental import pallas as pl


def kernel(obs, edge_index, act, W1, b1, W2, b2, Wm1, bm1, Wm2, bm2, Wmu, bmu, log_std):
    raise NotImplementedError("write your pallas kernel here")


# speedup vs baseline: 6.9460x; 6.9460x over previous
import functools

import jax
import jax.numpy as jnp
from jax import lax
from jax.experimental import pallas as pl
from jax.experimental.pallas import tpu as pltpu
from jax.experimental.pallas import tpu_sc as plsc

NC = 2
NS = 16
LANES = 16
CHUNK = 128


def _sc_aggregate(hl, src_r, dst_r, n_pad, with_deg):
    num_chunks = src_r.shape[1]
    d = hl.shape[1]
    rpt = n_pad // NS
    nz = rpt // CHUNK

    out_type = [jax.ShapeDtypeStruct((NC, n_pad, d), jnp.float32)]
    scratch = [
        pltpu.VMEM((num_chunks, CHUNK), jnp.int32),
        pltpu.VMEM((num_chunks, CHUNK), jnp.int32),
        pltpu.VMEM((CHUNK, d), jnp.float32),
        pltpu.VMEM_SHARED((n_pad, d), jnp.float32),
        pltpu.SemaphoreType.DMA,
    ]
    if with_deg:
        out_type.append(jax.ShapeDtypeStruct((NC, n_pad, LANES), jnp.float32))
        scratch += [
            pltpu.VMEM((CHUNK, LANES), jnp.float32),
            pltpu.VMEM_SHARED((n_pad, LANES), jnp.float32),
        ]

    mesh = plsc.VectorSubcoreMesh(core_axis_name="c", subcore_axis_name="s")

    def body(hl_hbm, src_hbm, dst_hbm, agg_out, *rest):
        if with_deg:
            deg_out, src_v, dst_v, rows_v, agg_sh, sem, ones_v, deg_sh = rest
        else:
            src_v, dst_v, rows_v, agg_sh, sem = rest
        c = lax.axis_index("c")
        s = lax.axis_index("s")
        wid = s * NC + c
        pltpu.sync_copy(src_hbm.at[wid], src_v)
        pltpu.sync_copy(dst_hbm.at[wid], dst_v)

        zero16 = jnp.zeros((LANES,), jnp.float32)
        base = s * rpt

        def _zero_rows(i, carry):
            for k in range(d // LANES):
                rows_v[i, pl.ds(k * LANES, LANES)] = zero16
            return carry

        lax.fori_loop(0, CHUNK, _zero_rows, 0)
        for q in range(nz):
            pltpu.sync_copy(rows_v, agg_sh.at[pl.ds(base + q * CHUNK, CHUNK)])

        if with_deg:
            def _zero_ones(i, carry):
                ones_v[i] = zero16
                return carry

            lax.fori_loop(0, CHUNK, _zero_ones, 0)
            for q in range(nz):
                pltpu.sync_copy(ones_v, deg_sh.at[pl.ds(base + q * CHUNK, CHUNK)])

            one16 = jnp.ones((LANES,), jnp.float32)

            def _fill_ones(i, carry):
                ones_v[i] = one16
                return carry

            lax.fori_loop(0, CHUNK, _fill_ones, 0)

        plsc.subcore_barrier()

        def _edge_chunk(j, carry):
            pltpu.async_copy(hl_hbm.at[src_v.at[j]], rows_v, sem).wait()
            pltpu.sync_copy(rows_v, agg_sh.at[dst_v.at[j]], add=True)
            if with_deg:
                pltpu.sync_copy(ones_v, deg_sh.at[dst_v.at[j]], add=True)
            return carry

        lax.fori_loop(0, num_chunks, _edge_chunk, 0)
        plsc.subcore_barrier()

        pltpu.sync_copy(agg_sh.at[pl.ds(base, rpt)],
                        agg_out.at[c, pl.ds(base, rpt)])
        if with_deg:
            pltpu.sync_copy(deg_sh.at[pl.ds(base, rpt)],
                            deg_out.at[c, pl.ds(base, rpt)])

    fn = pl.kernel(body, out_type=out_type, mesh=mesh, scratch_types=scratch,
                   compiler_params=pltpu.CompilerParams(
                       use_tc_tiling_on_sc=False))
    return fn(hl, src_r, dst_r)


def _tc_linear1(h, w, b):
    def body(h_ref, w_ref, b_ref, o_ref):
        o_ref[...] = (
            jnp.dot(h_ref[...], w_ref[...], preferred_element_type=jnp.float32)
            + b_ref[...]
        )

    return pl.pallas_call(
        body,
        out_shape=jax.ShapeDtypeStruct((h.shape[0], w.shape[1]), jnp.float32),
    )(h, w, b.reshape(1, -1))


def _tc_linear2(hl, p, d0, d1, w, b, gnn):
    def body(hl_ref, p_ref, d0_ref, d1_ref, w_ref, b_ref, o_ref):
        deg = jnp.maximum(d0_ref[...] + d1_ref[...], 1.0)
        a = jnp.maximum(hl_ref[...], 0.0)
        g = jnp.maximum((p_ref[0] + p_ref[1]) / deg, 0.0)
        wv = w_ref[...]
        o_ref[...] = (
            jnp.dot(a, wv[:gnn], preferred_element_type=jnp.float32)
            + jnp.dot(g, wv[gnn:], preferred_element_type=jnp.float32)
            + b_ref[...]
        )

    return pl.pallas_call(
        body,
        out_shape=jax.ShapeDtypeStruct((hl.shape[0], w.shape[1]), jnp.float32),
    )(hl, p, d0, d1, w, b.reshape(1, -1))


def _tc_head(hl3, p3, d3, w1r, b1, w2, b2, wmu, bmu, act2, log_std,
             gnn, phases):
    ag = hl3.shape[0]
    adim = wmu.shape[1]

    def body(hl_ref, p_ref, d_ref, w1_ref, b1_ref, w2_ref, b2_ref,
             wmu_ref, bmu_ref, act_ref, ls_ref, mu_ref, lp_ref):
        acc = jnp.zeros((ag, w2_ref.shape[0]), jnp.float32)
        for p in range(phases):
            deg = jnp.maximum(d_ref[:, p, 0:1] + d_ref[:, p, 1:2], 1.0)
            a_p = jnp.maximum(hl_ref[:, p, :], 0.0)
            g_p = jnp.maximum((p_ref[0, :, p, :] + p_ref[1, :, p, :]) / deg,
                              0.0)
            acc = acc + jnp.dot(a_p, w1_ref[p, :gnn],
                                preferred_element_type=jnp.float32)
            acc = acc + jnp.dot(g_p, w1_ref[p, gnn:],
                                preferred_element_type=jnp.float32)
        x = jnp.tanh(acc + b1_ref[...])
        x = jnp.tanh(jnp.dot(x, w2_ref[...],
                             preferred_element_type=jnp.float32) + b2_ref[...])
        mu = jnp.dot(x, wmu_ref[...],
                     preferred_element_type=jnp.float32) + bmu_ref[...]
        ls = ls_ref[...]
        z = (act_ref[...] - mu) * jnp.exp(-ls)
        lp = jnp.sum(-0.5 * z * z - ls - 0.5 * jnp.log(2.0 * jnp.pi),
                     axis=1, keepdims=True)
        mu_ref[...] = mu
        lp_ref[...] = lp

    return pl.pallas_call(
        body,
        out_shape=(
            jax.ShapeDtypeStruct((ag, adim), jnp.float32),
            jax.ShapeDtypeStruct((ag, 1), jnp.float32),
        ),
    )(hl3, p3, d3, w1r, b1.reshape(1, -1), w2, b2.reshape(1, -1),
      wmu, bmu.reshape(1, -1), act2, log_std.reshape(1, -1))


def kernel(obs, edge_index, act, W1, b1, W2, b2, Wm1, bm1, Wm2, bm2,
           Wmu, bmu, log_std):
    steps, nodes_per_step, feats = obs.shape
    n = steps * nodes_per_step
    gnn = W1.shape[1]
    e = edge_index.shape[1]
    phases = Wm1.shape[0] // (2 * gnn)
    ag = n // phases
    adim = Wmu.shape[1]
    nw = NC * NS

    ep = -(-e // (nw * CHUNK)) * (nw * CHUNK)
    n_pad = -(-n // (NS * CHUNK)) * (NS * CHUNK)
    if n_pad == n and ep > e:
        n_pad += NS * CHUNK
    src = edge_index[0]
    dst = edge_index[1]
    if ep > e:
        src = jnp.concatenate([src, jnp.zeros((ep - e,), jnp.int32)])
        dst = jnp.concatenate([dst, jnp.full((ep - e,), n, jnp.int32)])
    src_r = src.reshape(nw, -1, CHUNK)
    dst_r = dst.reshape(nw, -1, CHUNK)

    h = obs.reshape(n, feats)
    hl1 = _tc_linear1(h, W1, b1)
    agg1, deg = _sc_aggregate(hl1, src_r, dst_r, n_pad, True)
    d0 = deg[0, :n, 0:1]
    d1 = deg[1, :n, 0:1]
    hl2 = _tc_linear2(hl1, agg1[:, :n], d0, d1, W2, b2, gnn)
    (agg2,) = _sc_aggregate(hl2, src_r, dst_r, n_pad, False)

    hl3 = hl2.reshape(ag, phases, gnn)
    p3 = agg2[:, :n].reshape(NC, ag, phases, gnn)
    d3 = jnp.concatenate([d0, d1], axis=1).reshape(ag, phases, 2)
    w1r = Wm1.reshape(phases, 2 * gnn, Wm1.shape[1])
    act2 = act.reshape(ag, adim)
    mu, logp = _tc_head(hl3, p3, d3, w1r, bm1, Wm2, bm2, Wmu, bmu, act2,
                        log_std, gnn, phases)
    return (mu.reshape(act.shape), logp.reshape(act.shape[:-1]))
